# baseline (speedup 1.0000x reference)
import jax
import jax.numpy as jnp
from jax.experimental import pallas as pl
from jax.experimental.pallas import tpu as pltpu

B, T, N, D = 128, 24, 100, 64
F = 2 * D + 1
RARITY_ALPHA = 0.5
BB = B // 2


def _step_kernel(obs_ref, mav_ref, len_ref,
                 rarw_ref, pg1_ref, wg2t_ref, bg2_ref,
                 wru_ref, wc_ref, bru_ref, bc_ref,
                 out_ref, h_ref, off_ref, roff_ref, vt_ref, eyeb_ref,
                 eyen_ref):
    t = pl.program_id(0)

    @pl.when(t == 0)
    def _init():
        pg = jax.lax.dot_general(
            jnp.maximum(pg1_ref[...], 0.0), wg2t_ref[...],
            (((1,), (1,)), ((), ())),
            preferred_element_type=jnp.float32) + bg2_ref[...]
        nrm = jnp.sqrt(jnp.sum(pg * pg, axis=-1, keepdims=True))
        vn = pg / jnp.maximum(nrm, 1e-12)
        logits = jax.lax.dot_general(
            vn, vn, (((1,), (1,)), ((), ())),
            preferred_element_type=jnp.float32)
        mx = jnp.max(logits, axis=-1, keepdims=True)
        e = jnp.exp(logits - mx)
        adj = e / jnp.sum(e, axis=-1, keepdims=True)
        ri = jax.lax.broadcasted_iota(jnp.int32, (N, N), 0)
        ci = jax.lax.broadcasted_iota(jnp.int32, (N, N), 1)
        noteye = (ri != ci).astype(jnp.float32)
        off = adj * noteye
        off_ref[...] = off.astype(jnp.bfloat16)
        roff_ref[...] = (off * rarw_ref[...]).astype(jnp.bfloat16)
        eyen_ref[...] = (ri == ci).astype(jnp.bfloat16)
        bi = jax.lax.broadcasted_iota(jnp.int32, (B, B), 0)
        bj = jax.lax.broadcasted_iota(jnp.int32, (B, B), 1)
        eyeb_ref[...] = (bi == bj).astype(jnp.bfloat16)
        vt_ref[...] = jnp.sum(
            (mav_ref[...] > 0.0).astype(jnp.float32), axis=1)
        h_ref[...] = jnp.zeros_like(h_ref)

    eyeb = eyeb_ref[...]
    eyen = eyen_ref[...]

    packed_nb = mav_ref[:, t, :]
    mask_nb = (packed_nb > 0.0).astype(jnp.bfloat16)
    avg_nb = jnp.abs(packed_nb) - 1.0
    rs_nb = (RARITY_ALPHA
             * jnp.tanh(avg_nb / (vt_ref[...] + 1.0))).astype(jnp.bfloat16)
    msrs = jax.lax.dot_general(
        jnp.concatenate([mask_nb, rs_nb], axis=1), eyen,
        (((0,), (0,)), ((), ())),
        preferred_element_type=jnp.float32)
    mask_f = msrs[:B, :].astype(jnp.bfloat16)
    rs_f = msrs[B:, :].astype(jnp.bfloat16)
    obs_nb = obs_ref[0].astype(jnp.bfloat16)
    end_f = (len_ref[...] - 1 == t)

    def gate(x, w_ref, b_ref):
        pre = jax.lax.dot_general(
            x, w_ref[...], (((2,), (1,)), ((1,), (0,))),
            preferred_element_type=jnp.float32)
        pre = jnp.transpose(pre, (1, 0, 2))
        return pre + b_ref[...][None]

    mask = mask_f
    rs = rs_f
    h = h_ref[:, :, :D]

    cur_obs = jax.lax.dot_general(
        eyeb, obs_nb, (((1,), (2,)), ((), ())),
        preferred_element_type=jnp.float32
    ).astype(jnp.bfloat16)

    diff = jnp.abs(rs[:, :, None] - rs[:, None, :])
    madj = mask[:, :, None] * mask[:, None, :]
    cur_adj = (off_ref[...][None] - roff_ref[...][None] * diff) \
        * madj + eyen[None]

    xh = jnp.concatenate([cur_obs, h, rs[:, :, None]], axis=-1)
    comb = jax.lax.dot_general(
        cur_adj, xh, (((2,), (1,)), ((0,), (0,))),
        preferred_element_type=jnp.float32
    ).astype(jnp.bfloat16)

    ru = jax.nn.sigmoid(gate(comb, wru_ref, bru_ref)).astype(jnp.bfloat16)
    r = ru[:, :, :D]
    u = ru[:, :, D:]

    m = mask[:, :, None]
    h_reset = h * (1.0 + m * (r - 1.0))
    xh_new = jnp.concatenate([cur_obs, h_reset, rs[:, :, None]], axis=-1)
    cand = jnp.tanh(gate(xh_new, wc_ref, bc_ref)).astype(jnp.bfloat16)
    mu = m * u
    h_next = h_reset * (1.0 - mu) + mu * cand

    snap = jnp.where(end_f[:, :, None], h_next, h_ref[:, :, D:])
    h_ref[...] = jnp.concatenate([h_next, snap], axis=-1)

    @pl.when(t == T - 1)
    def _emit():
        out_ref[...] = jax.lax.dot_general(
            h_ref[:, :, D:], eyeb, (((0,), (0,)), ((), ())),
            preferred_element_type=jnp.float32)


@jax.jit
def _run(obs_p, mav, lengths, rarity_W, pg1, Wg2t, bg2,
         Wru2, Wc2, bru2, bc2):
    grid = (T,)
    whole = []
    for w in (mav, lengths, rarity_W, pg1, Wg2t, bg2,
              Wru2, Wc2, bru2, bc2):
        whole.append(
            pl.BlockSpec(w.shape, lambda t, nd=w.ndim: (0,) * nd))
    specs = [pl.BlockSpec((1, N, D, B), lambda t: (t, 0, 0, 0))] + whole
    return pl.pallas_call(
        _step_kernel,
        grid=grid,
        in_specs=specs,
        out_specs=pl.BlockSpec((N, D, B), lambda t: (0, 0, 0)),
        out_shape=jax.ShapeDtypeStruct((N, D, B), jnp.float32),
        scratch_shapes=[
            pltpu.VMEM((B, N, 2 * D), jnp.bfloat16),
            pltpu.VMEM((N, N), jnp.bfloat16),
            pltpu.VMEM((N, N), jnp.bfloat16),
            pltpu.VMEM((N, B), jnp.float32),
            pltpu.VMEM((B, B), jnp.bfloat16),
            pltpu.VMEM((N, N), jnp.bfloat16),
        ],
        compiler_params=pltpu.CompilerParams(
            dimension_semantics=("arbitrary",),
        ),
    )(obs_p, mav, lengths, rarity_W, pg1, Wg2t, bg2,
      Wru2, Wc2, bru2, bc2)


def kernel(obs_emb, adj, observed_mask, observed_tp, tp_emb_tensor, lengths,
           avg_interval, var_prior_emb_tensor, rarity_W, Wg1, bg1, Wg2, bg2,
           Wu, bu, Wr, br, Wc, bc):
    del adj, observed_tp, tp_emb_tensor
    obs_p = obs_emb.transpose(1, 2, 3, 0)
    maskf = observed_mask.astype(jnp.float32)
    mav = ((avg_interval + 1.0) * (2.0 * maskf - 1.0)).transpose(2, 1, 0)

    def repack(w):
        w = w.astype(jnp.bfloat16)
        return jnp.concatenate(
            [w[:, :D, :], w[:, D + 1:, :], w[:, D:D + 1, :]], axis=1)

    Wru2 = jnp.concatenate([repack(Wr), repack(Wu)], axis=2)
    Wc2 = repack(Wc)
    bru2 = jnp.concatenate([br, bu], axis=1)
    pg1 = var_prior_emb_tensor @ Wg1 + bg1[None]
    out_p = _run(obs_p, mav, lengths, rarity_W, pg1,
                 Wg2.transpose(1, 0), bg2.reshape(1, -1),
                 Wru2, Wc2, bru2, bc)
    return out_p.transpose(2, 0, 1)


# speedup vs baseline: 1.1551x; 1.1551x over previous
import jax
import jax.numpy as jnp
from jax.experimental import pallas as pl
from jax.experimental.pallas import tpu as pltpu

B, T, N, D = 128, 24, 100, 64
F = 2 * D + 1
RARITY_ALPHA = 0.5
BB = B // 2


def _step_kernel(obs_ref, mav_ref, len_ref,
                 rarw_ref, pg1_ref, wg2t_ref, bg2_ref,
                 wru_ref, wc_ref, bru_ref, bc_ref,
                 out_ref, h_ref, off_ref, roff_ref, vt_ref, eyeb_ref,
                 eyen_ref, snap_ref):
    t = pl.program_id(0)

    @pl.when(t == 0)
    def _init():
        pg = jax.lax.dot_general(
            jnp.maximum(pg1_ref[...], 0.0), wg2t_ref[...],
            (((1,), (1,)), ((), ())),
            preferred_element_type=jnp.float32) + bg2_ref[...]
        nrm = jnp.sqrt(jnp.sum(pg * pg, axis=-1, keepdims=True))
        vn = pg / jnp.maximum(nrm, 1e-12)
        logits = jax.lax.dot_general(
            vn, vn, (((1,), (1,)), ((), ())),
            preferred_element_type=jnp.float32)
        mx = jnp.max(logits, axis=-1, keepdims=True)
        e = jnp.exp(logits - mx)
        adj = e / jnp.sum(e, axis=-1, keepdims=True)
        ri = jax.lax.broadcasted_iota(jnp.int32, (N, N), 0)
        ci = jax.lax.broadcasted_iota(jnp.int32, (N, N), 1)
        noteye = (ri != ci).astype(jnp.float32)
        off = adj * noteye
        off_ref[...] = off.astype(jnp.bfloat16)
        roff_ref[...] = (off * rarw_ref[...]).astype(jnp.bfloat16)
        eyen_ref[...] = (ri == ci).astype(jnp.bfloat16)
        bi = jax.lax.broadcasted_iota(jnp.int32, (B, B), 0)
        bj = jax.lax.broadcasted_iota(jnp.int32, (B, B), 1)
        eyeb_ref[...] = (bi == bj).astype(jnp.bfloat16)
        vt_ref[...] = jnp.sum(
            (mav_ref[...] > 0.0).astype(jnp.float32), axis=1)
        h_ref[...] = jnp.zeros_like(h_ref)
        snap_ref[...] = jnp.zeros_like(snap_ref)

    eyeb = eyeb_ref[...]
    eyen = eyen_ref[...]

    packed_nb = mav_ref[:, t, :]
    mask_nb = (packed_nb > 0.0).astype(jnp.bfloat16)
    avg_nb = jnp.abs(packed_nb) - 1.0
    rs_nb = (RARITY_ALPHA
             * jnp.tanh(avg_nb / (vt_ref[...] + 1.0))).astype(jnp.bfloat16)
    msrs = jax.lax.dot_general(
        jnp.concatenate([mask_nb, rs_nb], axis=1), eyen,
        (((0,), (0,)), ((), ())),
        preferred_element_type=jnp.float32)
    mask_f = msrs[:B, :].astype(jnp.bfloat16)
    rs_f = msrs[B:, :].astype(jnp.bfloat16)
    obs_nb = obs_ref[0].astype(jnp.bfloat16)
    end_f = (len_ref[...] - 1 == t)

    def gate(x, w_ref, b_ref):
        pre = jax.lax.dot_general(
            x, w_ref[...], (((2,), (1,)), ((1,), (0,))),
            preferred_element_type=jnp.float32)
        pre = jnp.transpose(pre, (1, 0, 2))
        return pre + b_ref[...][None]

    mask = mask_f
    rs = rs_f
    h = h_ref[...]

    cur_obs = jax.lax.dot_general(
        eyeb, obs_nb, (((1,), (2,)), ((), ())),
        preferred_element_type=jnp.float32
    ).astype(jnp.bfloat16)

    diff = jnp.abs(rs[:, :, None] - rs[:, None, :])
    madj = mask[:, :, None] * mask[:, None, :]
    cur_adj = (off_ref[...][None] - roff_ref[...][None] * diff) \
        * madj + eyen[None]

    xh = jnp.concatenate([cur_obs, h, rs[:, :, None]], axis=-1)
    comb = jax.lax.dot_general(
        cur_adj, xh, (((2,), (1,)), ((0,), (0,))),
        preferred_element_type=jnp.float32
    ).astype(jnp.bfloat16)

    ru = jax.nn.sigmoid(gate(comb, wru_ref, bru_ref)).astype(jnp.bfloat16)
    r = ru[:, :, :D]
    u = ru[:, :, D:]

    m = mask[:, :, None]
    h_reset = h * (1.0 + m * (r - 1.0))
    xh_new = jnp.concatenate([cur_obs, h_reset, rs[:, :, None]], axis=-1)
    cand = jnp.tanh(gate(xh_new, wc_ref, bc_ref)).astype(jnp.bfloat16)
    mu = m * u
    h_next = h_reset * (1.0 - mu) + mu * cand

    h_ref[...] = h_next
    snap_ref[...] = jnp.where(end_f[:, :, None], h_next, snap_ref[...])

    @pl.when(t == T - 1)
    def _emit():
        out_ref[...] = jax.lax.dot_general(
            snap_ref[...], eyeb, (((0,), (0,)), ((), ())),
            preferred_element_type=jnp.float32)


@jax.jit
def _run(obs_p, mav, lengths, rarity_W, pg1, Wg2t, bg2,
         Wru2, Wc2, bru2, bc2):
    grid = (T,)
    whole = []
    for w in (mav, lengths, rarity_W, pg1, Wg2t, bg2,
              Wru2, Wc2, bru2, bc2):
        whole.append(
            pl.BlockSpec(w.shape, lambda t, nd=w.ndim: (0,) * nd))
    specs = [pl.BlockSpec((1, N, D, B), lambda t: (t, 0, 0, 0))] + whole
    return pl.pallas_call(
        _step_kernel,
        grid=grid,
        in_specs=specs,
        out_specs=pl.BlockSpec((N, D, B), lambda t: (0, 0, 0)),
        out_shape=jax.ShapeDtypeStruct((N, D, B), jnp.float32),
        scratch_shapes=[
            pltpu.VMEM((B, N, D), jnp.bfloat16),
            pltpu.VMEM((N, N), jnp.bfloat16),
            pltpu.VMEM((N, N), jnp.bfloat16),
            pltpu.VMEM((N, B), jnp.float32),
            pltpu.VMEM((B, B), jnp.bfloat16),
            pltpu.VMEM((N, N), jnp.bfloat16),
            pltpu.VMEM((B, N, D), jnp.bfloat16),
        ],
        compiler_params=pltpu.CompilerParams(
            dimension_semantics=("arbitrary",),
        ),
    )(obs_p, mav, lengths, rarity_W, pg1, Wg2t, bg2,
      Wru2, Wc2, bru2, bc2)


def kernel(obs_emb, adj, observed_mask, observed_tp, tp_emb_tensor, lengths,
           avg_interval, var_prior_emb_tensor, rarity_W, Wg1, bg1, Wg2, bg2,
           Wu, bu, Wr, br, Wc, bc):
    del adj, observed_tp, tp_emb_tensor
    obs_p = obs_emb.transpose(1, 2, 3, 0)
    maskf = observed_mask.astype(jnp.float32)
    mav = ((avg_interval + 1.0) * (2.0 * maskf - 1.0)).transpose(2, 1, 0)

    def repack(w):
        w = w.astype(jnp.bfloat16)
        return jnp.concatenate(
            [w[:, :D, :], w[:, D + 1:, :], w[:, D:D + 1, :]], axis=1)

    Wru2 = jnp.concatenate([repack(Wr), repack(Wu)], axis=2)
    Wc2 = repack(Wc)
    bru2 = jnp.concatenate([br, bu], axis=1)
    pg1 = var_prior_emb_tensor @ Wg1 + bg1[None]
    out_p = _run(obs_p, mav, lengths, rarity_W, pg1,
                 Wg2.transpose(1, 0), bg2.reshape(1, -1),
                 Wru2, Wc2, bru2, bc)
    return out_p.transpose(2, 0, 1)
